# baseline (speedup 1.0000x reference)
import functools

import jax
import jax.numpy as jnp
from jax import lax
from jax.experimental import pallas as pl
from jax.experimental.pallas import tpu as pltpu
from jax.experimental.pallas import tpu_sc as plsc

_V = 131072
_D = 32
_B = 1024
_MB = 128
_VC = 32768


def _gather_sc(x, emb4):
    info = plsc.get_sparse_core_info()
    nw = info.num_cores * info.num_subcores
    b_per_w = _B // nw
    mesh = plsc.VectorSubcoreMesh(core_axis_name="c", subcore_axis_name="s")

    @functools.partial(
        pl.kernel,
        mesh=mesh,
        out_type=jax.ShapeDtypeStruct((_B, 128), jnp.float32),
        scratch_types=[
            pltpu.VMEM((b_per_w,), jnp.int32),
            pltpu.VMEM((b_per_w,), jnp.int32),
            pltpu.VMEM((b_per_w, 128), jnp.float32),
            pltpu.SemaphoreType.DMA,
        ],
    )
    def gather_kernel(idx_hbm, table_hbm, out_hbm, idx_v, q_v, rows_v, sem):
        wid = lax.axis_index("s") * info.num_cores + lax.axis_index("c")
        base = wid * b_per_w
        pltpu.sync_copy(idx_hbm.at[pl.ds(base, b_per_w)], idx_v)
        for i in range(b_per_w // 16):
            sl = pl.ds(i * 16, 16)
            q_v[sl] = lax.shift_right_logical(idx_v[sl], 2)
        pltpu.async_copy(table_hbm.at[q_v], rows_v, sem).wait()
        pltpu.sync_copy(rows_v, out_hbm.at[pl.ds(base, b_per_w)])

    return gather_kernel(x, emb4)


def _decode_body(g4_ref, oh_ref, wt_ref, enc_ref, dec_ref):
    g4 = g4_ref[...]
    oh = oh_ref[...]
    pre = g4[:, 0:_D] * oh[:, 0:1]
    for k in range(1, 4):
        pre += g4[:, k * _D:(k + 1) * _D] * oh[:, k:k + 1]
    enc = jnp.tanh(pre)
    enc_ref[...] = enc
    dec_ref[...] = lax.dot_general(
        enc, wt_ref[...], (((1,), (0,)), ((), ())),
        preferred_element_type=jnp.float32)


def _decode_tc(gathered4, onehot, w_t):
    return pl.pallas_call(
        _decode_body,
        grid=(_V // _VC, _B // _MB),
        in_specs=[
            pl.BlockSpec((_MB, 128), lambda c, m: (m, 0)),
            pl.BlockSpec((_MB, 4), lambda c, m: (m, 0)),
            pl.BlockSpec((_D, _VC), lambda c, m: (0, c)),
        ],
        out_specs=[
            pl.BlockSpec((_MB, _D), lambda c, m: (m, 0)),
            pl.BlockSpec((_MB, _VC), lambda c, m: (m, c)),
        ],
        out_shape=[
            jax.ShapeDtypeStruct((_B, _D), jnp.float32),
            jax.ShapeDtypeStruct((_B, _V), jnp.float32),
        ],
    )(gathered4, onehot, w_t)


def kernel(x, emb_table, W_dec):
    xi = x.astype(jnp.int32)
    emb4 = emb_table.reshape(_V // 4, 128)
    gathered4 = _gather_sc(xi, emb4)
    onehot = jax.nn.one_hot(jnp.bitwise_and(xi, 3), 4, dtype=jnp.float32)
    w_t = W_dec.T
    encoding, decoded = _decode_tc(gathered4, onehot, w_t)
    return (encoding, decoded)
